# Initial kernel scaffold; baseline (speedup 1.0000x reference)
import jax
import jax.numpy as jnp
from jax.experimental import pallas as pl


def kernel(edge_index, adj_values, features_data, user_emb, item_emb, Wq, Wk, Wv, Wf, bf, Wg0, bg0, Wg1, bg1, Wb0, bb0, Wb1, bb1):
    raise NotImplementedError("write your pallas kernel here")


# speedup vs baseline: 2.1031x; 2.1031x over previous
import functools

import jax
import jax.numpy as jnp
from jax import lax
from jax.experimental import pallas as pl
from jax.experimental.pallas import tpu as pltpu
from jax.experimental.pallas import tpu_sc as plsc

_NC = 2
_NS = 16
_L = 16


def _make_spmm(N, E, C):
    per_sub = E // _NS
    assert E % _NS == 0 and per_sub % C == 0
    n_chunks = per_sub // C
    ROWS_BIG = ((N + _NS - 1) // _NS + 7) // 8 * 8
    ROWS_LAST = N - ROWS_BIG * (_NS - 1)
    assert ROWS_LAST > 0 and ROWS_LAST % 8 == 0

    mesh = plsc.VectorSubcoreMesh(core_axis_name="c", subcore_axis_name="s")

    @functools.partial(
        pl.kernel,
        mesh=mesh,
        out_type=jax.ShapeDtypeStruct((_NC, N, 128), jnp.float32),
        scratch_types=[
            pltpu.VMEM((C, _L), jnp.float32),
            pltpu.VMEM((C,), jnp.int32),
            pltpu.VMEM((C,), jnp.int32),
            pltpu.VMEM((C, 128), jnp.float32),
            pltpu.VMEM_SHARED((N, 128), jnp.float32),
            pltpu.SemaphoreType.DMA,
        ],
    )
    def spmm(rows_hbm, cols_hbm, vals_hbm, ego_hbm, zeros_hbm, out_hbm,
             valb_v, colc_v, rowc_v, buf_v, accum, sem):
        c = lax.axis_index("c")
        s = lax.axis_index("s")

        @pl.when(s < _NS - 1)
        def _():
            pltpu.sync_copy(zeros_hbm, accum.at[pl.ds(s * ROWS_BIG, ROWS_BIG)])

        @pl.when(s == _NS - 1)
        def _():
            pltpu.sync_copy(zeros_hbm.at[pl.ds(0, ROWS_LAST)],
                            accum.at[pl.ds((_NS - 1) * ROWS_BIG, ROWS_LAST)])

        plsc.subcore_barrier()

        def chunk_body(g, carry):
            pltpu.sync_copy(cols_hbm.at[c, s, g], colc_v)
            pltpu.sync_copy(vals_hbm.at[s, g], valb_v)
            pltpu.async_copy(ego_hbm.at[colc_v], buf_v, sem).wait()

            def edge_body(r, carry2):
                vv = valb_v[r]
                for j in range(128 // _L):
                    buf_v[r, pl.ds(j * _L, _L)] = buf_v[r, pl.ds(j * _L, _L)] * vv
                return carry2

            lax.fori_loop(0, C, edge_body, 0)
            pltpu.sync_copy(rows_hbm.at[s, g], rowc_v)
            pltpu.sync_copy(buf_v, accum.at[rowc_v], add=True)
            return carry

        lax.fori_loop(0, n_chunks, chunk_body, 0)
        plsc.subcore_barrier()

        @pl.when(s < _NS - 1)
        def _():
            pltpu.sync_copy(accum.at[pl.ds(s * ROWS_BIG, ROWS_BIG)],
                            out_hbm.at[c, pl.ds(s * ROWS_BIG, ROWS_BIG)])

        @pl.when(s == _NS - 1)
        def _():
            pltpu.sync_copy(accum.at[pl.ds((_NS - 1) * ROWS_BIG, ROWS_LAST)],
                            out_hbm.at[c, pl.ds((_NS - 1) * ROWS_BIG, ROWS_LAST)])

    return spmm


def _attention(x, Wq, Wk, Wv, Wf, bf):
    F, D = x.shape
    scale = float(1.0 / (float(D) ** 0.5))

    def body(x_ref, wq_ref, wk_ref, wv_ref, wf_ref, bf_ref, out_ref):
        xv = x_ref[...]
        q = lax.dot(xv, wq_ref[...], precision=lax.Precision.HIGHEST)
        k = lax.dot(xv, wk_ref[...], precision=lax.Precision.HIGHEST)
        v = lax.dot(xv, wv_ref[...], precision=lax.Precision.HIGHEST)
        s = lax.dot_general(q, k, (((1,), (1,)), ((), ())),
                            precision=lax.Precision.HIGHEST) * scale
        m = jnp.max(s, axis=-1, keepdims=True)
        p = jnp.exp(s - m)
        p = p / jnp.sum(p, axis=-1, keepdims=True)
        av = lax.dot(p, v, precision=lax.Precision.HIGHEST)
        out_ref[...] = (lax.dot(av, wf_ref[...], precision=lax.Precision.HIGHEST)
                        + bf_ref[...])

    return pl.pallas_call(
        body,
        out_shape=jax.ShapeDtypeStruct((F, Wf.shape[1]), jnp.float32),
    )(x, Wq, Wk, Wv, Wf, bf.reshape(1, -1))


def _dense_layer(ego, side2, Wg, bg, Wb, bb, BN=1000):
    N, D = ego.shape
    assert N % BN == 0

    def body(side_ref, ego_ref, wg_ref, bg_ref, wb_ref, bb_ref,
             ego_out_ref, norm_out_ref):
        side = jnp.concatenate([side_ref[0], side_ref[1]], axis=-1)
        e = ego_ref[...]
        sum_emb = lax.dot(side, wg_ref[...],
                          precision=lax.Precision.HIGHEST) + bg_ref[...]
        sum_emb = jnp.where(sum_emb >= 0, sum_emb, sum_emb * 0.01)
        bi = lax.dot(e * side, wb_ref[...],
                     precision=lax.Precision.HIGHEST) + bb_ref[...]
        bi = jnp.where(bi >= 0, bi, bi * 0.01)
        out = sum_emb + bi
        nrm = jnp.sqrt(jnp.sum(out * out, axis=-1, keepdims=True))
        ego_out_ref[...] = out
        norm_out_ref[...] = out / jnp.maximum(nrm, 1e-12)

    grid = (N // BN,)
    return pl.pallas_call(
        body,
        grid=grid,
        in_specs=[
            pl.BlockSpec((_NC, BN, 128), lambda i: (0, i, 0)),
            pl.BlockSpec((BN, D), lambda i: (i, 0)),
            pl.BlockSpec((D, D), lambda i: (0, 0)),
            pl.BlockSpec((1, D), lambda i: (0, 0)),
            pl.BlockSpec((D, D), lambda i: (0, 0)),
            pl.BlockSpec((1, D), lambda i: (0, 0)),
        ],
        out_specs=[
            pl.BlockSpec((BN, D), lambda i: (i, 0)),
            pl.BlockSpec((BN, D), lambda i: (i, 0)),
        ],
        out_shape=[
            jax.ShapeDtypeStruct((N, D), jnp.float32),
            jax.ShapeDtypeStruct((N, D), jnp.float32),
        ],
    )(side2, ego, Wg, bg.reshape(1, -1), Wb, bb.reshape(1, -1))


def kernel(edge_index, adj_values, features_data, user_emb, item_emb,
           Wq, Wk, Wv, Wf, bf, Wg0, bg0, Wg1, bg1, Wb0, bb0, Wb1, bb1):
    n_users = user_emb.shape[0]
    n_items = item_emb.shape[0]
    n_feat = features_data.shape[0]
    N = n_users + n_items + n_feat
    D = user_emb.shape[1]
    E = adj_values.shape[0]
    C = 80
    per_sub = E // _NS
    n_chunks = per_sub // C

    atten_score = _attention(features_data, Wq, Wk, Wv, Wf, bf)
    ego = jnp.concatenate([user_emb, item_emb, atten_score], axis=0)

    row = edge_index[0].astype(jnp.int32)
    col = edge_index[1].astype(jnp.int32)
    rows3 = row.reshape(_NS, n_chunks, C)
    cols_half = jnp.stack([col * 2, col * 2 + 1], axis=0)
    cols4 = cols_half.reshape(_NC, _NS, n_chunks, C)
    vals2 = jnp.broadcast_to(adj_values[:, None], (E, _L)).reshape(
        _NS, n_chunks, C, _L)
    rows_big = ((N + _NS - 1) // _NS + 7) // 8 * 8
    zeros = jnp.zeros((rows_big, 128), jnp.float32)

    spmm = _make_spmm(N, E, C)

    all_embeddings = [ego]
    for (Wg, bg, Wb, bb) in ((Wg0, bg0, Wb0, bb0), (Wg1, bg1, Wb1, bb1)):
        side2 = spmm(rows3, cols4, vals2, ego.reshape(2 * N, 128), zeros)
        ego, normed = _dense_layer(ego, side2, Wg, bg, Wb, bb)
        all_embeddings.append(normed)

    all_cat = jnp.concatenate(all_embeddings, axis=1)
    u_g = all_cat[:n_users]
    i_g = all_cat[n_users:n_users + n_items]
    return (u_g, i_g, atten_score)


# speedup vs baseline: 5.1353x; 2.4417x over previous
import functools

import jax
import jax.numpy as jnp
from jax import lax
from jax.experimental import pallas as pl
from jax.experimental.pallas import tpu as pltpu
from jax.experimental.pallas import tpu_sc as plsc

_NC = 2
_NS = 16
_L = 16


def _make_spmm(N, E, C):
    per_sub = E // _NS
    assert E % _NS == 0 and per_sub % C == 0
    n_chunks = per_sub // C
    ROWS_BIG = ((N + _NS - 1) // _NS + 7) // 8 * 8
    ROWS_LAST = N - ROWS_BIG * (_NS - 1)
    assert ROWS_LAST > 0 and ROWS_LAST % 8 == 0

    mesh = plsc.VectorSubcoreMesh(core_axis_name="c", subcore_axis_name="s")

    @functools.partial(
        pl.kernel,
        mesh=mesh,
        out_type=jax.ShapeDtypeStruct((_NC, N, 128), jnp.float32),
        scratch_types=[
            pltpu.VMEM((per_sub,), jnp.int32),
            pltpu.VMEM((1, C), jnp.float32),
            pltpu.VMEM((1, C), jnp.float32),
            pltpu.VMEM((1, C), jnp.int32),
            pltpu.VMEM((1, C), jnp.int32),
            pltpu.VMEM((C, 128), jnp.float32),
            pltpu.VMEM((C, 128), jnp.float32),
            pltpu.VMEM_SHARED((N, 128), jnp.float32),
            pltpu.SemaphoreType.DMA,
            pltpu.SemaphoreType.DMA,
            pltpu.SemaphoreType.DMA,
            pltpu.SemaphoreType.DMA,
            pltpu.SemaphoreType.DMA,
            pltpu.SemaphoreType.DMA,
        ],
    )
    def spmm(rows_hbm, cols_hbm, vals_hbm, ego_hbm, zeros_hbm, out_hbm,
             cols_v, valc0, valc1, rowc0, rowc1, buf0_v, buf1_v, accum,
             gsem0, gsem1, vsem0, vsem1, rsem0, rsem1):
        c = lax.axis_index("c")
        s = lax.axis_index("s")
        bufs = (buf0_v, buf1_v)
        valcs = (valc0, valc1)
        rowcs = (rowc0, rowc1)
        gsems = (gsem0, gsem1)
        vsems = (vsem0, vsem1)
        rsems = (rsem0, rsem1)

        @pl.when(s < _NS - 1)
        def _():
            pltpu.sync_copy(zeros_hbm, accum.at[pl.ds(s * ROWS_BIG, ROWS_BIG)])

        @pl.when(s == _NS - 1)
        def _():
            pltpu.sync_copy(zeros_hbm.at[pl.ds(0, ROWS_LAST)],
                            accum.at[pl.ds((_NS - 1) * ROWS_BIG, ROWS_LAST)])

        pltpu.sync_copy(cols_hbm.at[c, s], cols_v)
        plsc.subcore_barrier()

        def start_fetch(j, b):
            pltpu.make_async_copy(
                ego_hbm.at[cols_v.at[pl.ds(j * C, C)]], bufs[b], gsems[b]).start()
            pltpu.make_async_copy(vals_hbm.at[s, j], valcs[b], vsems[b]).start()
            pltpu.make_async_copy(rows_hbm.at[s, j], rowcs[b], rsems[b]).start()

        def wait_fetch(j, b):
            pltpu.make_async_copy(
                ego_hbm.at[cols_v.at[pl.ds(j * C, C)]], bufs[b], gsems[b]).wait()
            pltpu.make_async_copy(vals_hbm.at[s, j], valcs[b], vsems[b]).wait()
            pltpu.make_async_copy(rows_hbm.at[s, j], rowcs[b], rsems[b]).wait()

        start_fetch(0, 0)

        def process_chunk(j, b, last=False):
            buf = bufs[b]
            valc = valcs[b]
            wait_fetch(j, b)

            if not last:
                @pl.when(j + 1 < n_chunks)
                def _():
                    start_fetch(j + 1, 1 - b)

            def group_body(g16, carry2):
                v16 = valc[0, pl.ds(g16 * _L, _L)]
                r0 = g16 * _L
                for r16 in range(_L):
                    vv = lax.gather(
                        v16, jnp.full((_L, 1), r16, jnp.int32),
                        lax.GatherDimensionNumbers(
                            offset_dims=(), collapsed_slice_dims=(0,),
                            start_index_map=(0,)),
                        slice_sizes=(1,),
                        mode=lax.GatherScatterMode.PROMISE_IN_BOUNDS)
                    r = r0 + r16
                    for k in range(128 // _L):
                        buf[r, pl.ds(k * _L, _L)] = (
                            buf[r, pl.ds(k * _L, _L)] * vv)
                return carry2

            lax.fori_loop(0, C // _L, group_body, 0)
            pltpu.sync_copy(buf, accum.at[rowcs[b].at[0]], add=True)

        def pair_body(h, carry):
            for b in range(2):
                process_chunk(2 * h + b, b)
            return carry

        lax.fori_loop(0, n_chunks // 2, pair_body, 0)
        if n_chunks % 2:
            process_chunk(n_chunks - 1, (n_chunks - 1) % 2, last=True)
        plsc.subcore_barrier()

        @pl.when(s < _NS - 1)
        def _():
            pltpu.sync_copy(accum.at[pl.ds(s * ROWS_BIG, ROWS_BIG)],
                            out_hbm.at[c, pl.ds(s * ROWS_BIG, ROWS_BIG)])

        @pl.when(s == _NS - 1)
        def _():
            pltpu.sync_copy(accum.at[pl.ds((_NS - 1) * ROWS_BIG, ROWS_LAST)],
                            out_hbm.at[c, pl.ds((_NS - 1) * ROWS_BIG, ROWS_LAST)])

    return spmm


def _attention(x, Wq, Wk, Wv, Wf, bf):
    F, D = x.shape
    scale = float(1.0 / (float(D) ** 0.5))

    def body(x_ref, wq_ref, wk_ref, wv_ref, wf_ref, bf_ref, out_ref):
        xv = x_ref[...]
        q = lax.dot(xv, wq_ref[...], precision=lax.Precision.HIGHEST)
        k = lax.dot(xv, wk_ref[...], precision=lax.Precision.HIGHEST)
        v = lax.dot(xv, wv_ref[...], precision=lax.Precision.HIGHEST)
        s = lax.dot_general(q, k, (((1,), (1,)), ((), ())),
                            precision=lax.Precision.HIGHEST) * scale
        m = jnp.max(s, axis=-1, keepdims=True)
        p = jnp.exp(s - m)
        p = p / jnp.sum(p, axis=-1, keepdims=True)
        av = lax.dot(p, v, precision=lax.Precision.HIGHEST)
        out_ref[...] = (lax.dot(av, wf_ref[...], precision=lax.Precision.HIGHEST)
                        + bf_ref[...])

    return pl.pallas_call(
        body,
        out_shape=jax.ShapeDtypeStruct((F, Wf.shape[1]), jnp.float32),
    )(x, Wq, Wk, Wv, Wf, bf.reshape(1, -1))


def _dense_layer(ego, side2, Wg, bg, Wb, bb, BN=1000):
    N, D = ego.shape
    assert N % BN == 0

    def body(side_ref, ego_ref, wg_ref, bg_ref, wb_ref, bb_ref,
             ego_out_ref, norm_out_ref):
        side = jnp.concatenate([side_ref[0], side_ref[1]], axis=-1)
        e = ego_ref[...]
        sum_emb = lax.dot(side, wg_ref[...],
                          precision=lax.Precision.HIGHEST) + bg_ref[...]
        sum_emb = jnp.where(sum_emb >= 0, sum_emb, sum_emb * 0.01)
        bi = lax.dot(e * side, wb_ref[...],
                     precision=lax.Precision.HIGHEST) + bb_ref[...]
        bi = jnp.where(bi >= 0, bi, bi * 0.01)
        out = sum_emb + bi
        nrm = jnp.sqrt(jnp.sum(out * out, axis=-1, keepdims=True))
        ego_out_ref[...] = out
        norm_out_ref[...] = out / jnp.maximum(nrm, 1e-12)

    grid = (N // BN,)
    return pl.pallas_call(
        body,
        grid=grid,
        in_specs=[
            pl.BlockSpec((_NC, BN, 128), lambda i: (0, i, 0)),
            pl.BlockSpec((BN, D), lambda i: (i, 0)),
            pl.BlockSpec((D, D), lambda i: (0, 0)),
            pl.BlockSpec((1, D), lambda i: (0, 0)),
            pl.BlockSpec((D, D), lambda i: (0, 0)),
            pl.BlockSpec((1, D), lambda i: (0, 0)),
        ],
        out_specs=[
            pl.BlockSpec((BN, D), lambda i: (i, 0)),
            pl.BlockSpec((BN, D), lambda i: (i, 0)),
        ],
        out_shape=[
            jax.ShapeDtypeStruct((N, D), jnp.float32),
            jax.ShapeDtypeStruct((N, D), jnp.float32),
        ],
    )(side2, ego, Wg, bg.reshape(1, -1), Wb, bb.reshape(1, -1))


def kernel(edge_index, adj_values, features_data, user_emb, item_emb,
           Wq, Wk, Wv, Wf, bf, Wg0, bg0, Wg1, bg1, Wb0, bb0, Wb1, bb1):
    n_users = user_emb.shape[0]
    n_items = item_emb.shape[0]
    n_feat = features_data.shape[0]
    N = n_users + n_items + n_feat
    D = user_emb.shape[1]
    E = adj_values.shape[0]
    C = 80
    per_sub = E // _NS
    n_chunks = per_sub // C

    atten_score = _attention(features_data, Wq, Wk, Wv, Wf, bf)
    ego = jnp.concatenate([user_emb, item_emb, atten_score], axis=0)

    row = edge_index[0].astype(jnp.int32)
    col = edge_index[1].astype(jnp.int32)
    rows3 = row.reshape(_NS, n_chunks, 1, C)
    cols_half = jnp.stack([col * 2, col * 2 + 1], axis=0)
    cols3 = cols_half.reshape(_NC, _NS, per_sub)
    vals3 = adj_values.reshape(_NS, n_chunks, 1, C)
    rows_big = ((N + _NS - 1) // _NS + 7) // 8 * 8
    zeros = jnp.zeros((rows_big, 128), jnp.float32)

    spmm = _make_spmm(N, E, C)

    all_embeddings = [ego]
    for (Wg, bg, Wb, bb) in ((Wg0, bg0, Wb0, bb0), (Wg1, bg1, Wb1, bb1)):
        side2 = spmm(rows3, cols3, vals3, ego.reshape(2 * N, 128), zeros)
        ego, normed = _dense_layer(ego, side2, Wg, bg, Wb, bb)
        all_embeddings.append(normed)

    all_cat = jnp.concatenate(all_embeddings, axis=1)
    u_g = all_cat[:n_users]
    i_g = all_cat[n_users:n_users + n_items]
    return (u_g, i_g, atten_score)


# speedup vs baseline: 5.8639x; 1.1419x over previous
import functools

import jax
import jax.numpy as jnp
from jax import lax
from jax.experimental import pallas as pl
from jax.experimental.pallas import tpu as pltpu
from jax.experimental.pallas import tpu_sc as plsc

_NC = 2
_NS = 16
_L = 16


def _make_spmm(N, E, C):
    per_sub = E // _NS
    assert E % _NS == 0 and per_sub % C == 0
    n_chunks = per_sub // C
    ROWS_BIG = ((N + _NS - 1) // _NS + 7) // 8 * 8
    ROWS_LAST = N - ROWS_BIG * (_NS - 1)
    assert ROWS_LAST > 0 and ROWS_LAST % 8 == 0

    mesh = plsc.VectorSubcoreMesh(core_axis_name="c", subcore_axis_name="s")

    @functools.partial(
        pl.kernel,
        mesh=mesh,
        out_type=jax.ShapeDtypeStruct((_NC, N, 128), jnp.float32),
        scratch_types=[
            pltpu.VMEM((per_sub,), jnp.int32),
            pltpu.VMEM((1, C), jnp.float32),
            pltpu.VMEM((1, C), jnp.float32),
            pltpu.VMEM((1, C), jnp.int32),
            pltpu.VMEM((1, C), jnp.int32),
            pltpu.VMEM((C, 128), jnp.float32),
            pltpu.VMEM((C, 128), jnp.float32),
            pltpu.VMEM_SHARED((N, 128), jnp.float32),
            pltpu.SemaphoreType.DMA,
            pltpu.SemaphoreType.DMA,
            pltpu.SemaphoreType.DMA,
            pltpu.SemaphoreType.DMA,
            pltpu.SemaphoreType.DMA,
            pltpu.SemaphoreType.DMA,
        ],
    )
    def spmm(rows_hbm, cols_hbm, vals_hbm, ego_hbm, zeros_hbm, out_hbm,
             cols_v, valc0, valc1, rowc0, rowc1, buf0_v, buf1_v, accum,
             gsem0, gsem1, vsem0, vsem1, rsem0, rsem1):
        c = lax.axis_index("c")
        s = lax.axis_index("s")
        bufs = (buf0_v, buf1_v)
        valcs = (valc0, valc1)
        rowcs = (rowc0, rowc1)
        gsems = (gsem0, gsem1)
        vsems = (vsem0, vsem1)
        rsems = (rsem0, rsem1)
        ego_c = ego_hbm.at[c]

        @pl.when(s < _NS - 1)
        def _():
            pltpu.sync_copy(zeros_hbm, accum.at[pl.ds(s * ROWS_BIG, ROWS_BIG)])

        @pl.when(s == _NS - 1)
        def _():
            pltpu.sync_copy(zeros_hbm.at[pl.ds(0, ROWS_LAST)],
                            accum.at[pl.ds((_NS - 1) * ROWS_BIG, ROWS_LAST)])

        pltpu.sync_copy(cols_hbm.at[s], cols_v)
        plsc.subcore_barrier()

        def start_fetch(j, b):
            pltpu.make_async_copy(
                ego_c.at[cols_v.at[pl.ds(j * C, C)]], bufs[b], gsems[b]).start()
            pltpu.make_async_copy(vals_hbm.at[s, j], valcs[b], vsems[b]).start()
            pltpu.make_async_copy(rows_hbm.at[s, j], rowcs[b], rsems[b]).start()

        def wait_fetch(j, b):
            pltpu.make_async_copy(
                ego_c.at[cols_v.at[pl.ds(j * C, C)]], bufs[b], gsems[b]).wait()
            pltpu.make_async_copy(vals_hbm.at[s, j], valcs[b], vsems[b]).wait()
            pltpu.make_async_copy(rows_hbm.at[s, j], rowcs[b], rsems[b]).wait()

        start_fetch(0, 0)

        def process_chunk(j, b, last=False):
            buf = bufs[b]
            valc = valcs[b]
            wait_fetch(j, b)

            if not last:
                @pl.when(j + 1 < n_chunks)
                def _():
                    start_fetch(j + 1, 1 - b)

            def group_body(g16, carry2):
                v16 = valc[0, pl.ds(g16 * _L, _L)]
                r0 = g16 * _L
                for r16 in range(_L):
                    vv = lax.gather(
                        v16, jnp.full((_L, 1), r16, jnp.int32),
                        lax.GatherDimensionNumbers(
                            offset_dims=(), collapsed_slice_dims=(0,),
                            start_index_map=(0,)),
                        slice_sizes=(1,),
                        mode=lax.GatherScatterMode.PROMISE_IN_BOUNDS)
                    r = r0 + r16
                    for k in range(128 // _L):
                        buf[r, pl.ds(k * _L, _L)] = (
                            buf[r, pl.ds(k * _L, _L)] * vv)
                return carry2

            lax.fori_loop(0, C // _L, group_body, 0)
            pltpu.sync_copy(buf, accum.at[rowcs[b].at[0]], add=True)

        def pair_body(h, carry):
            for b in range(2):
                process_chunk(2 * h + b, b)
            return carry

        lax.fori_loop(0, n_chunks // 2, pair_body, 0)
        if n_chunks % 2:
            process_chunk(n_chunks - 1, (n_chunks - 1) % 2, last=True)
        plsc.subcore_barrier()

        @pl.when(s < _NS - 1)
        def _():
            pltpu.sync_copy(accum.at[pl.ds(s * ROWS_BIG, ROWS_BIG)],
                            out_hbm.at[c, pl.ds(s * ROWS_BIG, ROWS_BIG)])

        @pl.when(s == _NS - 1)
        def _():
            pltpu.sync_copy(accum.at[pl.ds((_NS - 1) * ROWS_BIG, ROWS_LAST)],
                            out_hbm.at[c, pl.ds((_NS - 1) * ROWS_BIG, ROWS_LAST)])

    return spmm


def _attention(x, Wq, Wk, Wv, Wf, bf):
    F, D = x.shape
    scale = float(1.0 / (float(D) ** 0.5))

    def body(x_ref, wq_ref, wk_ref, wv_ref, wf_ref, bf_ref, out_ref):
        xv = x_ref[...]
        q = lax.dot(xv, wq_ref[...], precision=lax.Precision.HIGHEST)
        k = lax.dot(xv, wk_ref[...], precision=lax.Precision.HIGHEST)
        v = lax.dot(xv, wv_ref[...], precision=lax.Precision.HIGHEST)
        s = lax.dot_general(q, k, (((1,), (1,)), ((), ())),
                            precision=lax.Precision.HIGHEST) * scale
        m = jnp.max(s, axis=-1, keepdims=True)
        p = jnp.exp(s - m)
        p = p / jnp.sum(p, axis=-1, keepdims=True)
        av = lax.dot(p, v, precision=lax.Precision.HIGHEST)
        out_ref[...] = (lax.dot(av, wf_ref[...], precision=lax.Precision.HIGHEST)
                        + bf_ref[...])

    return pl.pallas_call(
        body,
        out_shape=jax.ShapeDtypeStruct((F, Wf.shape[1]), jnp.float32),
    )(x, Wq, Wk, Wv, Wf, bf.reshape(1, -1))


def _leaky(x):
    return jnp.where(x >= 0, x, x * 0.01)


def _layer_math(side_ref, e, wg_ref, bg_ref, wb_ref, bb_ref):
    side = jnp.concatenate([side_ref[0], side_ref[1]], axis=-1)
    sum_emb = _leaky(lax.dot(side, wg_ref[...]) + bg_ref[...])
    bi = _leaky(lax.dot(e * side, wb_ref[...]) + bb_ref[...])
    out = sum_emb + bi
    nrm = jnp.sqrt(jnp.sum(out * out, axis=-1, keepdims=True))
    return out, out / jnp.maximum(nrm, 1e-12)


def _dense_layer1(ego, side2, Wg, bg, Wb, bb, BN=1000):
    N, D = ego.shape
    assert N % BN == 0

    def body(side_ref, ego_ref, wg_ref, bg_ref, wb_ref, bb_ref,
             ego2_out_ref, ego_out_ref, norm_out_ref):
        out, normed = _layer_math(side_ref, ego_ref[...],
                                  wg_ref, bg_ref, wb_ref, bb_ref)
        ego2_out_ref[0] = out[:, :128]
        ego2_out_ref[1] = out[:, 128:]
        ego_out_ref[...] = out
        norm_out_ref[...] = normed

    grid = (N // BN,)
    return pl.pallas_call(
        body,
        grid=grid,
        in_specs=[
            pl.BlockSpec((_NC, BN, 128), lambda i: (0, i, 0)),
            pl.BlockSpec((BN, D), lambda i: (i, 0)),
            pl.BlockSpec((D, D), lambda i: (0, 0)),
            pl.BlockSpec((1, D), lambda i: (0, 0)),
            pl.BlockSpec((D, D), lambda i: (0, 0)),
            pl.BlockSpec((1, D), lambda i: (0, 0)),
        ],
        out_specs=[
            pl.BlockSpec((_NC, BN, 128), lambda i: (0, i, 0)),
            pl.BlockSpec((BN, D), lambda i: (i, 0)),
            pl.BlockSpec((BN, D), lambda i: (i, 0)),
        ],
        out_shape=[
            jax.ShapeDtypeStruct((_NC, N, 128), jnp.float32),
            jax.ShapeDtypeStruct((N, D), jnp.float32),
            jax.ShapeDtypeStruct((N, D), jnp.float32),
        ],
    )(side2, ego, Wg, bg.reshape(1, -1), Wb, bb.reshape(1, -1))


def _dense_out(ego0, n1, ego1, side2, Wg, bg, Wb, bb, row_off, rows, BN):
    N, D = ego1.shape
    assert rows % BN == 0 and row_off % BN == 0
    off = row_off // BN

    def body(side_ref, ego1_ref, ego0_ref, n1_ref,
             wg_ref, bg_ref, wb_ref, bb_ref, out_ref):
        _, normed = _layer_math(side_ref, ego1_ref[...],
                                wg_ref, bg_ref, wb_ref, bb_ref)
        out_ref[...] = jnp.concatenate(
            [ego0_ref[...], n1_ref[...], normed], axis=-1)

    grid = (rows // BN,)
    return pl.pallas_call(
        body,
        grid=grid,
        in_specs=[
            pl.BlockSpec((_NC, BN, 128), lambda i: (0, i + off, 0)),
            pl.BlockSpec((BN, D), lambda i: (i + off, 0)),
            pl.BlockSpec((BN, D), lambda i: (i + off, 0)),
            pl.BlockSpec((BN, D), lambda i: (i + off, 0)),
            pl.BlockSpec((D, D), lambda i: (0, 0)),
            pl.BlockSpec((1, D), lambda i: (0, 0)),
            pl.BlockSpec((D, D), lambda i: (0, 0)),
            pl.BlockSpec((1, D), lambda i: (0, 0)),
        ],
        out_specs=pl.BlockSpec((BN, 3 * D), lambda i: (i, 0)),
        out_shape=jax.ShapeDtypeStruct((rows, 3 * D), jnp.float32),
    )(side2, ego1, ego0, n1, Wg, bg.reshape(1, -1), Wb, bb.reshape(1, -1))


def kernel(edge_index, adj_values, features_data, user_emb, item_emb,
           Wq, Wk, Wv, Wf, bf, Wg0, bg0, Wg1, bg1, Wb0, bb0, Wb1, bb1):
    n_users = user_emb.shape[0]
    n_items = item_emb.shape[0]
    n_feat = features_data.shape[0]
    N = n_users + n_items + n_feat
    D = user_emb.shape[1]
    E = adj_values.shape[0]
    C = 80
    per_sub = E // _NS
    n_chunks = per_sub // C

    atten_score = _attention(features_data, Wq, Wk, Wv, Wf, bf)
    ego0 = jnp.concatenate([user_emb, item_emb, atten_score], axis=0)
    ego0_2 = jnp.stack([ego0[:, :128], ego0[:, 128:]], axis=0)

    row = edge_index[0].astype(jnp.int32)
    col = edge_index[1].astype(jnp.int32)
    rows3 = row.reshape(_NS, n_chunks, 1, C)
    cols2 = col.reshape(_NS, per_sub)
    vals3 = adj_values.reshape(_NS, n_chunks, 1, C)
    rows_big = ((N + _NS - 1) // _NS + 7) // 8 * 8
    zeros = jnp.zeros((rows_big, 128), jnp.float32)

    spmm = _make_spmm(N, E, C)

    side2_1 = spmm(rows3, cols2, vals3, ego0_2, zeros)
    ego1_2, ego1, n1 = _dense_layer1(ego0, side2_1, Wg0, bg0, Wb0, bb0)
    side2_2 = spmm(rows3, cols2, vals3, ego1_2, zeros)
    u_g = _dense_out(ego0, n1, ego1, side2_2, Wg1, bg1, Wb1, bb1,
                     0, n_users, 1000)
    i_g = _dense_out(ego0, n1, ego1, side2_2, Wg1, bg1, Wb1, bb1,
                     n_users, n_items, 200)
    return (u_g, i_g, atten_score)
